# Initial kernel scaffold; baseline (speedup 1.0000x reference)
import jax
import jax.numpy as jnp
from jax.experimental import pallas as pl


def kernel(x, adj, W0, b0, W1, b1, Wg1, bg1, Wg2, bg2):
    raise NotImplementedError("write your pallas kernel here")


# speedup vs baseline: 59.6520x; 59.6520x over previous
import functools

import jax
import jax.numpy as jnp
from jax.experimental import pallas as pl

N = 4096
R = 512
GRID = N // R
F32 = jnp.float32


def _k1_deg_v1(adj_ref, x_ref, w0_ref, v1_ref, d1_ref, cs_ref):
    blk = adj_ref[...]
    rs = jnp.sum(blk, axis=1, keepdims=True)
    d1 = (rs + 1.0) ** -0.5
    d1_ref[...] = d1
    i = pl.program_id(0)

    @pl.when(i == 0)
    def _():
        cs_ref[...] = jnp.zeros_like(cs_ref)

    cs_ref[...] += jnp.sum(blk, axis=0, keepdims=True)
    xw = jnp.dot(x_ref[...], w0_ref[...], preferred_element_type=F32)
    v1_ref[...] = d1 * xw


def _k2_spmm1(adj_ref, v1f_ref, v1b_ref, d1_ref, b0_ref, w1_ref, v2_ref):
    acc = jnp.dot(adj_ref[...], v1f_ref[...], preferred_element_type=F32)
    acc = acc + v1b_ref[...]
    h1 = jax.nn.relu(d1_ref[...] * acc + b0_ref[...])
    v2_ref[...] = d1_ref[...] * jnp.dot(h1, w1_ref[...],
                                        preferred_element_type=F32)


def _k3_spmm2(adj_ref, v2f_ref, v2b_ref, d1_ref, d2_ref, b1_ref, wg1_ref,
              logits_ref, v3_ref):
    acc = jnp.dot(adj_ref[...], v2f_ref[...], preferred_element_type=F32)
    acc = acc + v2b_ref[...]
    logits = d1_ref[...] * acc + b1_ref[...]
    logits_ref[...] = logits
    v3_ref[...] = d2_ref[...] * jnp.dot(logits, wg1_ref[...],
                                        preferred_element_type=F32)


def _k4_spmmT1(adj_ref, v3f_ref, v3b_ref, d2_ref, bg1_ref, wg2_ref, v4_ref):
    acc = jax.lax.dot_general(adj_ref[...], v3f_ref[...],
                              (((0,), (0,)), ((), ())),
                              preferred_element_type=F32)
    acc = acc + v3b_ref[...]
    t = jax.nn.relu(d2_ref[...] * acc + bg1_ref[...])
    v4_ref[...] = d2_ref[...] * jnp.dot(t, wg2_ref[...],
                                        preferred_element_type=F32)


def _k5_spmmT2(adj_ref, v4f_ref, v4b_ref, d2_ref, bg2_ref, logits_ref,
               out_ref):
    acc = jax.lax.dot_general(adj_ref[...], v4f_ref[...],
                              (((0,), (0,)), ((), ())),
                              preferred_element_type=F32)
    acc = acc + v4b_ref[...]
    t2 = d2_ref[...] * acc + bg2_ref[...]
    t3 = jnp.log(jnp.exp(t2) + 1.1)
    o = logits_ref[...] * t3
    m = jnp.max(o, axis=1, keepdims=True)
    lse = m + jnp.log(jnp.sum(jnp.exp(o - m), axis=1, keepdims=True))
    out_ref[...] = o - lse


def _row_blk(f):
    return pl.BlockSpec((R, f), lambda i: (i, 0))


def _full(n, f):
    return pl.BlockSpec((n, f), lambda i: (0, 0))


@jax.jit
def kernel(x, adj, W0, b0, W1, b1, Wg1, bg1, Wg2, bg2):
    D = x.shape[1]
    H = W0.shape[1]
    C = W1.shape[1]
    b0r, b1r = b0[None, :], b1[None, :]
    bg1r, bg2r = bg1[None, :], bg2[None, :]

    v1, d1, cs = pl.pallas_call(
        _k1_deg_v1,
        grid=(GRID,),
        in_specs=[_row_blk(N), _row_blk(D), _full(D, H)],
        out_specs=[_row_blk(H), _row_blk(1), _full(1, N)],
        out_shape=[jax.ShapeDtypeStruct((N, H), F32),
                   jax.ShapeDtypeStruct((N, 1), F32),
                   jax.ShapeDtypeStruct((1, N), F32)],
    )(adj, x, W0)

    d2 = (cs.reshape(N, 1) + 1.0) ** -0.5

    v2 = pl.pallas_call(
        _k2_spmm1,
        grid=(GRID,),
        in_specs=[_row_blk(N), _full(N, H), _row_blk(H), _row_blk(1),
                  _full(1, H), _full(H, C)],
        out_specs=_row_blk(C),
        out_shape=jax.ShapeDtypeStruct((N, C), F32),
    )(adj, v1, v1, d1, b0r, W1)

    logits, v3 = pl.pallas_call(
        _k3_spmm2,
        grid=(GRID,),
        in_specs=[_row_blk(N), _full(N, C), _row_blk(C), _row_blk(1),
                  _row_blk(1), _full(1, C), _full(C, C)],
        out_specs=[_row_blk(C), _row_blk(C)],
        out_shape=[jax.ShapeDtypeStruct((N, C), F32),
                   jax.ShapeDtypeStruct((N, C), F32)],
    )(adj, v2, v2, d1, d2, b1r, Wg1)

    col_strip = pl.BlockSpec((N, R), lambda j: (0, j))

    v4 = pl.pallas_call(
        _k4_spmmT1,
        grid=(GRID,),
        in_specs=[col_strip, _full(N, C), _row_blk(C), _row_blk(1),
                  _full(1, C), _full(C, C)],
        out_specs=_row_blk(C),
        out_shape=jax.ShapeDtypeStruct((N, C), F32),
    )(adj, v3, v3, d2, bg1r, Wg2)

    out = pl.pallas_call(
        _k5_spmmT2,
        grid=(GRID,),
        in_specs=[col_strip, _full(N, C), _row_blk(C), _row_blk(1),
                  _full(1, C), _row_blk(C)],
        out_specs=_row_blk(C),
        out_shape=jax.ShapeDtypeStruct((N, C), F32),
    )(adj, v4, v4, d2, bg2r, logits)

    return out


# speedup vs baseline: 78.0496x; 1.3084x over previous
import functools

import jax
import jax.numpy as jnp
from jax.experimental import pallas as pl

N = 4096
R = 512
GRID = N // R
F32 = jnp.float32


def _k1_deg_v1(adj_ref, x_ref, w0_ref, v1_ref, d1_ref, cs_ref, adj8_ref):
    blk = adj_ref[...]
    adj8_ref[...] = blk.astype(jnp.int8)
    rs = jnp.sum(blk, axis=1, keepdims=True)
    d1 = (rs + 1.0) ** -0.5
    d1_ref[...] = d1
    i = pl.program_id(0)

    @pl.when(i == 0)
    def _():
        cs_ref[...] = jnp.zeros_like(cs_ref)

    cs_ref[...] += jnp.sum(blk, axis=0, keepdims=True)
    xw = jnp.dot(x_ref[...], w0_ref[...], preferred_element_type=F32)
    v1_ref[...] = d1 * xw


def _k2_spmm1(adj_ref, v1f_ref, v1b_ref, d1_ref, b0_ref, w1_ref, v2_ref):
    acc = jnp.dot(adj_ref[...].astype(F32), v1f_ref[...],
                  preferred_element_type=F32)
    acc = acc + v1b_ref[...]
    h1 = jax.nn.relu(d1_ref[...] * acc + b0_ref[...])
    v2_ref[...] = d1_ref[...] * jnp.dot(h1, w1_ref[...],
                                        preferred_element_type=F32)


def _k3_spmm2(adj_ref, v2f_ref, v2b_ref, d1_ref, d2_ref, b1_ref, wg1_ref,
              logits_ref, v3_ref):
    acc = jnp.dot(adj_ref[...].astype(F32), v2f_ref[...],
                  preferred_element_type=F32)
    acc = acc + v2b_ref[...]
    logits = d1_ref[...] * acc + b1_ref[...]
    logits_ref[...] = logits
    v3_ref[...] = d2_ref[...] * jnp.dot(logits, wg1_ref[...],
                                        preferred_element_type=F32)


def _k4_spmmT1(adj_ref, v3f_ref, v3b_ref, d2_ref, bg1_ref, wg2_ref, v4_ref):
    acc = jax.lax.dot_general(adj_ref[...].astype(F32), v3f_ref[...],
                              (((0,), (0,)), ((), ())),
                              preferred_element_type=F32)
    acc = acc + v3b_ref[...]
    t = jax.nn.relu(d2_ref[...] * acc + bg1_ref[...])
    v4_ref[...] = d2_ref[...] * jnp.dot(t, wg2_ref[...],
                                        preferred_element_type=F32)


def _k5_spmmT2(adj_ref, v4f_ref, v4b_ref, d2_ref, bg2_ref, logits_ref,
               out_ref):
    acc = jax.lax.dot_general(adj_ref[...].astype(F32), v4f_ref[...],
                              (((0,), (0,)), ((), ())),
                              preferred_element_type=F32)
    acc = acc + v4b_ref[...]
    t2 = d2_ref[...] * acc + bg2_ref[...]
    t3 = jnp.log(jnp.exp(t2) + 1.1)
    o = logits_ref[...] * t3
    m = jnp.max(o, axis=1, keepdims=True)
    lse = m + jnp.log(jnp.sum(jnp.exp(o - m), axis=1, keepdims=True))
    out_ref[...] = o - lse


def _row_blk(f):
    return pl.BlockSpec((R, f), lambda i: (i, 0))


def _full(n, f):
    return pl.BlockSpec((n, f), lambda i: (0, 0))


@jax.jit
def kernel(x, adj, W0, b0, W1, b1, Wg1, bg1, Wg2, bg2):
    D = x.shape[1]
    H = W0.shape[1]
    C = W1.shape[1]
    b0r, b1r = b0[None, :], b1[None, :]
    bg1r, bg2r = bg1[None, :], bg2[None, :]

    v1, d1, cs, adj8 = pl.pallas_call(
        _k1_deg_v1,
        grid=(GRID,),
        in_specs=[_row_blk(N), _row_blk(D), _full(D, H)],
        out_specs=[_row_blk(H), _row_blk(1), _full(1, N), _row_blk(N)],
        out_shape=[jax.ShapeDtypeStruct((N, H), F32),
                   jax.ShapeDtypeStruct((N, 1), F32),
                   jax.ShapeDtypeStruct((1, N), F32),
                   jax.ShapeDtypeStruct((N, N), jnp.int8)],
    )(adj, x, W0)

    d2 = (cs.reshape(N, 1) + 1.0) ** -0.5

    v2 = pl.pallas_call(
        _k2_spmm1,
        grid=(GRID,),
        in_specs=[_row_blk(N), _full(N, H), _row_blk(H), _row_blk(1),
                  _full(1, H), _full(H, C)],
        out_specs=_row_blk(C),
        out_shape=jax.ShapeDtypeStruct((N, C), F32),
    )(adj8, v1, v1, d1, b0r, W1)

    logits, v3 = pl.pallas_call(
        _k3_spmm2,
        grid=(GRID,),
        in_specs=[_row_blk(N), _full(N, C), _row_blk(C), _row_blk(1),
                  _row_blk(1), _full(1, C), _full(C, C)],
        out_specs=[_row_blk(C), _row_blk(C)],
        out_shape=[jax.ShapeDtypeStruct((N, C), F32),
                   jax.ShapeDtypeStruct((N, C), F32)],
    )(adj8, v2, v2, d1, d2, b1r, Wg1)

    col_strip = pl.BlockSpec((N, R), lambda j: (0, j))

    v4 = pl.pallas_call(
        _k4_spmmT1,
        grid=(GRID,),
        in_specs=[col_strip, _full(N, C), _row_blk(C), _row_blk(1),
                  _full(1, C), _full(C, C)],
        out_specs=_row_blk(C),
        out_shape=jax.ShapeDtypeStruct((N, C), F32),
    )(adj8, v3, v3, d2, bg1r, Wg2)

    out = pl.pallas_call(
        _k5_spmmT2,
        grid=(GRID,),
        in_specs=[col_strip, _full(N, C), _row_blk(C), _row_blk(1),
                  _full(1, C), _row_blk(C)],
        out_specs=_row_blk(C),
        out_shape=jax.ShapeDtypeStruct((N, C), F32),
    )(adj8, v4, v4, d2, bg2r, logits)

    return out
